# Initial kernel scaffold; baseline (speedup 1.0000x reference)
import jax
import jax.numpy as jnp
from jax.experimental import pallas as pl


def kernel(z, pos, e, n):
    raise NotImplementedError("write your pallas kernel here")


# speedup vs baseline: 71.0763x; 71.0763x over previous
import functools

import jax
import jax.numpy as jnp
from jax import lax
from jax.experimental import pallas as pl
from jax.experimental.pallas import tpu as pltpu
from jax.experimental.pallas import tpu_sc as plsc

_LANES = 16
_NUM_WORKERS = 32


def _ceil_to(x: int, m: int) -> int:
    return ((x + m - 1) // m) * m


@functools.lru_cache(maxsize=None)
def _make_ids_kernel(num_mols: int, padded_total: int):
    chunk = padded_total // _NUM_WORKERS
    assert chunk % _LANES == 0 and chunk % 8 == 0
    n_pad = _ceil_to(num_mols, _LANES)
    n_vecs = n_pad // _LANES
    c_vecs = chunk // _LANES

    mesh = plsc.VectorSubcoreMesh(core_axis_name="c", subcore_axis_name="s")

    @functools.partial(
        pl.kernel,
        mesh=mesh,
        compiler_params=pltpu.CompilerParams(needs_layout_passes=False),
        out_type=jax.ShapeDtypeStruct((padded_total,), jnp.int32),
        scratch_types=[
            pltpu.VMEM((n_pad,), jnp.int32),
            pltpu.VMEM((chunk,), jnp.int32),
        ],
    )
    def ids_kernel(n_hbm, out_hbm, n_v, marks_v):
        wid = lax.axis_index("s") * 2 + lax.axis_index("c")
        start = wid * chunk

        pltpu.sync_copy(n_hbm, n_v)

        zeros16 = jnp.zeros((_LANES,), jnp.int32)

        def zero_body(i, _):
            marks_v[pl.ds(i * _LANES, _LANES)] = zeros16
            return 0

        lax.fori_loop(0, c_vecs, zero_body, 0, unroll=4)

        lane_iota = lax.iota(jnp.int32, _LANES)
        ones16 = jnp.ones((_LANES,), jnp.int32)

        def scan_body(i, carry):
            run, base = carry
            m_idx = i * _LANES + lane_iota
            v = n_v[pl.ds(i * _LANES, _LANES)]
            incl = jnp.cumsum(v) + run
            valid = m_idx < (num_mols - 1)
            j = incl - start
            in_chunk = valid & (j >= 1) & (j < chunk)
            j_safe = jnp.clip(j, 0, chunk - 1)
            plsc.store_scatter(marks_v, [j_safe], ones16, mask=in_chunk)
            base = base + jnp.sum(jnp.where(valid & (incl <= start), 1, 0))
            run = run + jnp.sum(v)
            return run, base

        _, base_id = lax.fori_loop(
            0, n_vecs, scan_body, (jnp.int32(0), jnp.int32(0))
        )

        def psum_body(i, carry):
            m = marks_v[pl.ds(i * _LANES, _LANES)]
            marks_v[pl.ds(i * _LANES, _LANES)] = jnp.cumsum(m) + carry
            return carry + jnp.sum(m)

        lax.fori_loop(0, c_vecs, psum_body, base_id)

        pltpu.sync_copy(marks_v, out_hbm.at[pl.ds(start, chunk)])

    return ids_kernel


def kernel(z, pos, e, n):
    num_mols = n.shape[0]
    total = pos.shape[0]
    padded_total = _ceil_to(total, _NUM_WORKERS * _LANES)
    n_pad = _ceil_to(num_mols, _LANES)
    ids_fn = _make_ids_kernel(num_mols, padded_total)
    n_in = n
    if n_pad != num_mols:
        n_in = jnp.pad(n, (0, n_pad - num_mols))
    batch_ids = ids_fn(n_in)[:total]
    return (z, pos, batch_ids, e.reshape(-1, 1))


# speedup vs baseline: 81.6787x; 1.1492x over previous
import functools

import jax
import jax.numpy as jnp
from jax import lax
from jax.experimental import pallas as pl
from jax.experimental.pallas import tpu as pltpu
from jax.experimental.pallas import tpu_sc as plsc

_LANES = 16
_NUM_WORKERS = 32


def _ceil_to(x: int, m: int) -> int:
    return ((x + m - 1) // m) * m


@functools.lru_cache(maxsize=None)
def _make_ids_kernel(num_mols: int, total: int):
    chunk = _ceil_to(total, _NUM_WORKERS * _LANES) // _NUM_WORKERS
    tail = total - (_NUM_WORKERS - 1) * chunk
    assert 0 < tail <= chunk and chunk % _LANES == 0 and chunk % 8 == 0
    n_pad = _ceil_to(num_mols, _LANES)
    n_vecs = n_pad // _LANES
    c_vecs = chunk // _LANES

    mesh = plsc.VectorSubcoreMesh(core_axis_name="c", subcore_axis_name="s")

    @functools.partial(
        pl.kernel,
        mesh=mesh,
        compiler_params=pltpu.CompilerParams(needs_layout_passes=False),
        out_type=jax.ShapeDtypeStruct((total,), jnp.int32),
        scratch_types=[
            pltpu.VMEM((n_pad,), jnp.int32),
            pltpu.VMEM((chunk,), jnp.int32),
        ],
    )
    def ids_kernel(n_hbm, out_hbm, n_v, marks_v):
        wid = lax.axis_index("s") * 2 + lax.axis_index("c")
        start = wid * chunk

        pltpu.sync_copy(n_hbm, n_v)

        zeros16 = jnp.zeros((_LANES,), jnp.int32)

        def zero_body(i, _):
            marks_v[pl.ds(i * _LANES, _LANES)] = zeros16
            return 0

        lax.fori_loop(0, c_vecs, zero_body, 0, unroll=8)

        lane_iota = lax.iota(jnp.int32, _LANES)
        ones16 = jnp.ones((_LANES,), jnp.int32)

        def scan_body(i, carry):
            run, base_acc = carry
            m_idx = i * _LANES + lane_iota
            v = n_v[pl.ds(i * _LANES, _LANES)]
            incl = jnp.cumsum(v) + run
            valid = m_idx < (num_mols - 1)
            j = incl - start
            in_chunk = valid & (j >= 1) & (j < chunk)
            j_safe = jnp.clip(j, 0, chunk - 1)
            plsc.store_scatter(marks_v, [j_safe], ones16, mask=in_chunk)
            base_acc = base_acc + jnp.where(valid & (incl <= start), 1, 0)
            run = run + jnp.sum(v)
            return run, base_acc

        _, base_acc = lax.fori_loop(
            0, n_vecs, scan_body, (jnp.int32(0), zeros16), unroll=8
        )
        base_id = jnp.sum(base_acc)

        def psum_body(i, carry):
            m = marks_v[pl.ds(i * _LANES, _LANES)]
            marks_v[pl.ds(i * _LANES, _LANES)] = jnp.cumsum(m) + carry
            return carry + jnp.sum(m)

        lax.fori_loop(0, c_vecs, psum_body, base_id, unroll=8)

        if tail == chunk:
            pltpu.sync_copy(marks_v, out_hbm.at[pl.ds(start, chunk)])
        else:

            @pl.when(wid < _NUM_WORKERS - 1)
            def _():
                pltpu.sync_copy(marks_v, out_hbm.at[pl.ds(start, chunk)])

            @pl.when(wid == _NUM_WORKERS - 1)
            def _():
                pltpu.sync_copy(
                    marks_v.at[pl.ds(0, tail)], out_hbm.at[pl.ds(start, tail)]
                )

    return ids_kernel


def kernel(z, pos, e, n):
    num_mols = n.shape[0]
    total = pos.shape[0]
    n_pad = _ceil_to(num_mols, _LANES)
    ids_fn = _make_ids_kernel(num_mols, total)
    n_in = n
    if n_pad != num_mols:
        n_in = jnp.pad(n, (0, n_pad - num_mols))
    batch_ids = ids_fn(n_in)
    return (z, pos, batch_ids, e.reshape(-1, 1))
